# baseline (speedup 1.0000x reference)
import functools

import jax
import jax.numpy as jnp
from jax import lax
from jax.experimental import pallas as pl
from jax.experimental.pallas import tpu as pltpu
from jax.experimental.pallas import tpu_sc as plsc

_B, _A, _C, _G = 8, 49104, 80, 20
_NTILES = 32
_CHUNK = 1536
_LCHUNK = _A - 31 * _CHUNK
_NVEC = _CHUNK // 16
_LNVEC = _LCHUNK // 16
_NGT = _B * _G
_CBLK = 8
_NCB = _C // _CBLK
_LN2 = 0.6931471805599453
_SQRT2 = 1.4142135623730951

_LOG_COEFS = (
    0.11589569107111188, -0.1885243878963612, 0.20655334597565636,
    -0.24886378324342587, 0.3329959787175392, -0.5000199301348872,
    1.0000040901688678, 6.432101458397899e-08,
)

_SC_PARAMS = pltpu.CompilerParams(use_tc_tiling_on_sc=False,
                                  needs_layout_passes=False)


def _mesh():
  return plsc.VectorSubcoreMesh(core_axis_name="c", subcore_axis_name="s",
                                num_cores=2, num_subcores=16)


def _sc_log(x):
  bits = plsc.bitcast(x, jnp.int32)
  e = ((bits >> 23) & 0xFF) - 127
  m = plsc.bitcast((bits & 0x7FFFFF) | 0x3F800000, jnp.float32)
  big = m > _SQRT2
  m = jnp.where(big, m * 0.5, m)
  ef = (e + jnp.where(big, 1, 0)).astype(jnp.float32)
  z = m - 1.0
  p = jnp.full_like(x, _LOG_COEFS[0])
  for c in _LOG_COEFS[1:]:
    p = p * z + c
  return ef * _LN2 + p


def _smooth_l1(d):
  d = jnp.abs(d)
  return jnp.where(d <= 1.0 / 9.0, 0.5 * 9.0 * d * d, d - 0.5 / 9.0)


def _tile_id():
  return lax.axis_index("s") * 2 + lax.axis_index("c")


def _assign_body(anch_hbm, gts_hbm, gti_hbm,
                 code_hbm, bidx_hbm, npos_hbm,
                 anch_v, gts_v, gti_v, code_v, bidx_v, acc_v):
  wid = _tile_id()
  last = wid == _NTILES - 1
  base = wid * _CHUNK

  @pl.when(jnp.logical_not(last))
  def _stage_full():
    pltpu.sync_copy(anch_hbm.at[:, pl.ds(base, _CHUNK)],
                    anch_v.at[:, pl.ds(0, _CHUNK)])

  @pl.when(last)
  def _stage_last():
    pltpu.sync_copy(anch_hbm.at[:, pl.ds(31 * _CHUNK, _LCHUNK)],
                    anch_v.at[:, pl.ds(0, _LCHUNK)])

  pltpu.sync_copy(gts_hbm, gts_v)
  pltpu.sync_copy(gti_hbm, gti_v)

  for b in range(_B):
    acc_v[b] = jnp.zeros((16,), jnp.float32)

  def process(o):
    a_x0 = anch_v[0, pl.ds(o, 16)]
    a_y0 = anch_v[1, pl.ds(o, 16)]
    a_x1 = anch_v[2, pl.ds(o, 16)]
    a_y1 = anch_v[3, pl.ds(o, 16)]
    a_area = (a_x1 - a_x0) * (a_y1 - a_y0)

    for b in range(_B):
      binter = jnp.full((16,), -1.0, jnp.float32)
      bdenom = jnp.full((16,), 1.0, jnp.float32)
      bidx = jnp.full((16,), b * _G, jnp.int32)
      for g in range(_G):
        k = b * _G + g
        gx0 = gts_v[k * 8 + 0]
        gy0 = gts_v[k * 8 + 1]
        gx1 = gts_v[k * 8 + 2]
        gy1 = gts_v[k * 8 + 3]
        gar = gts_v[k * 8 + 4]
        iw = jnp.maximum(
            jnp.minimum(a_x1, gx1) - jnp.maximum(a_x0, gx0), 0.0)
        ih = jnp.maximum(
            jnp.minimum(a_y1, gy1) - jnp.maximum(a_y0, gy0), 0.0)
        inter = iw * ih
        denom = (a_area + gar) - inter
        m = inter * bdenom > binter * denom
        binter = jnp.where(m, inter, binter)
        bdenom = jnp.where(m, denom, bdenom)
        bidx = jnp.where(m, jnp.full((16,), k, jnp.int32), bidx)

      pos = binter >= 0.5 * bdenom
      neg = binter < 0.4 * bdenom
      lab = plsc.load_gather(gti_v, [bidx])
      code = jnp.where(pos, lab, jnp.where(neg, -2, -1))
      code_v[b, pl.ds(o, 16)] = code
      bidx_v[b, pl.ds(o, 16)] = jnp.where(pos, bidx, -1)
      acc_v[b] = acc_v[b] + jnp.where(pos, 1.0, 0.0)

  npair = jnp.where(last, _LNVEC // 2, _NVEC // 2)

  def body(i, carry):
    process(i * 32)
    process(i * 32 + 16)
    return carry

  lax.fori_loop(0, npair, body, 0)

  @pl.when(last)
  def _odd_tail():
    process((_LNVEC - 1) * 16)

  @pl.when(jnp.logical_not(last))
  def _out_full():
    for b in range(_B):
      pltpu.sync_copy(code_v.at[b, pl.ds(0, _CHUNK)],
                      code_hbm.at[b, pl.ds(base, _CHUNK)])
      pltpu.sync_copy(bidx_v.at[b, pl.ds(0, _CHUNK)],
                      bidx_hbm.at[b, pl.ds(base, _CHUNK)])

  @pl.when(last)
  def _out_last():
    for b in range(_B):
      pltpu.sync_copy(code_v.at[b, pl.ds(0, _LCHUNK)],
                      code_hbm.at[b, pl.ds(31 * _CHUNK, _LCHUNK)])
      pltpu.sync_copy(bidx_v.at[b, pl.ds(0, _LCHUNK)],
                      bidx_hbm.at[b, pl.ds(31 * _CHUNK, _LCHUNK)])

  pltpu.sync_copy(acc_v, npos_hbm.at[wid])


def _sc_assign(anch_t, gtsf, gti):
  fn = pl.kernel(
      _assign_body,
      out_type=(
          jax.ShapeDtypeStruct((_B, _A), jnp.int32),
          jax.ShapeDtypeStruct((_B, _A), jnp.int32),
          jax.ShapeDtypeStruct((_NTILES, _B, 16), jnp.float32),
      ),
      mesh=_mesh(),
      scratch_types=[
          pltpu.VMEM((4, _CHUNK), jnp.float32),
          pltpu.VMEM((_NGT * 8, 16), jnp.float32),
          pltpu.VMEM((_NGT,), jnp.int32),
          pltpu.VMEM((_B, _CHUNK), jnp.int32),
          pltpu.VMEM((_B, _CHUNK), jnp.int32),
          pltpu.VMEM((_B, 16), jnp.float32),
      ],
      compiler_params=_SC_PARAMS,
      name="assign",
  )
  return fn(anch_t, gtsf, gti)


def _regloss_body(anch_hbm, reg_hbm, bidx_hbm, gtf_hbm,
                  part_hbm,
                  anch_v, reg_v, bidx_v, gtf_v, acc_v):
  wid = _tile_id()
  last = wid == _NTILES - 1
  base = wid * _CHUNK

  @pl.when(jnp.logical_not(last))
  def _stage_full():
    pltpu.sync_copy(anch_hbm.at[:, pl.ds(base, _CHUNK)],
                    anch_v.at[:, pl.ds(0, _CHUNK)])
    pltpu.sync_copy(reg_hbm.at[:, pl.ds(base, _CHUNK)],
                    reg_v.at[:, pl.ds(0, _CHUNK)])
    for b in range(_B):
      pltpu.sync_copy(bidx_hbm.at[b, pl.ds(base, _CHUNK)],
                      bidx_v.at[b, pl.ds(0, _CHUNK)])

  @pl.when(last)
  def _stage_last():
    pltpu.sync_copy(anch_hbm.at[:, pl.ds(31 * _CHUNK, _LCHUNK)],
                    anch_v.at[:, pl.ds(0, _LCHUNK)])
    pltpu.sync_copy(reg_hbm.at[:, pl.ds(31 * _CHUNK, _LCHUNK)],
                    reg_v.at[:, pl.ds(0, _LCHUNK)])
    for b in range(_B):
      pltpu.sync_copy(bidx_hbm.at[b, pl.ds(31 * _CHUNK, _LCHUNK)],
                      bidx_v.at[b, pl.ds(0, _LCHUNK)])

  pltpu.sync_copy(gtf_hbm, gtf_v)

  for b in range(_B):
    acc_v[b] = jnp.zeros((16,), jnp.float32)

  col = [jnp.full((16,), c, jnp.int32) for c in range(4)]
  nvec = jnp.where(last, _LNVEC, _NVEC)

  def body(i, carry):
    o = i * 16
    a_x0 = anch_v[0, pl.ds(o, 16)]
    a_y0 = anch_v[1, pl.ds(o, 16)]
    a_x1 = anch_v[2, pl.ds(o, 16)]
    a_y1 = anch_v[3, pl.ds(o, 16)]
    aw = jnp.abs(a_x0 - a_x1)
    ah = jnp.abs(a_y0 - a_y1)
    actr_x = a_x0 + 0.5 * aw
    actr_y = a_y0 + 0.5 * ah
    inv_aw = 1.0 / aw
    inv_ah = 1.0 / ah

    for b in range(_B):
      bidx = bidx_v[b, pl.ds(o, 16)]
      pos = bidx >= 0
      idx = jnp.maximum(bidx, 0)
      gx0 = plsc.load_gather(gtf_v, [col[0], idx])
      gy0 = plsc.load_gather(gtf_v, [col[1], idx])
      gx1 = plsc.load_gather(gtf_v, [col[2], idx])
      gy1 = plsc.load_gather(gtf_v, [col[3], idx])
      gw0 = gx1 - gx0
      gh0 = gy1 - gy0
      gcx = gx0 + 0.5 * gw0
      gcy = gy0 + 0.5 * gh0
      gw = jnp.maximum(gw0, 1.0)
      gh = jnp.maximum(gh0, 1.0)
      tdx = (gcx - actr_x) * inv_aw
      tdy = (gcy - actr_y) * inv_ah
      tdw = _sc_log(gw * inv_aw)
      tdh = _sc_log(gh * inv_ah)
      r0 = reg_v[b, pl.ds(o, 16)]
      r1 = reg_v[8 + b, pl.ds(o, 16)]
      r2 = reg_v[16 + b, pl.ds(o, 16)]
      r3 = reg_v[24 + b, pl.ds(o, 16)]
      rl = (_smooth_l1(tdx - r0) + _smooth_l1(tdy - r1)
            + _smooth_l1(tdh - r2) + _smooth_l1(tdw - r3))
      acc_v[b] = acc_v[b] + jnp.where(pos, rl, 0.0)
    return carry

  lax.fori_loop(0, nvec, body, 0)
  pltpu.sync_copy(acc_v, part_hbm.at[wid])


def _sc_regloss(anch_t, reg32, bidx, gtf):
  fn = pl.kernel(
      _regloss_body,
      out_type=jax.ShapeDtypeStruct((_NTILES, _B, 16), jnp.float32),
      mesh=_mesh(),
      scratch_types=[
          pltpu.VMEM((4, _CHUNK), jnp.float32),
          pltpu.VMEM((32, _CHUNK), jnp.float32),
          pltpu.VMEM((_B, _CHUNK), jnp.int32),
          pltpu.VMEM((8, _NGT), jnp.float32),
          pltpu.VMEM((_B, 16), jnp.float32),
      ],
      compiler_params=_SC_PARAMS,
      name="regloss",
  )
  return fn(anch_t, reg32, bidx, gtf)


_FCH = 1024
_NFCH = (_A + _FCH - 1) // _FCH


def _focal_body(cls_ref, code_ref, out_ref):
  ci = pl.program_id(1)
  cls_id_full = (ci * _CBLK
                 + lax.broadcasted_iota(jnp.int32, (_CBLK, _FCH), 0))
  acc = jnp.zeros((_CBLK, 128), jnp.float32)
  for j in range(_NFCH):
    lo = j * _FCH
    wch = min(_FCH, _A - lo)
    x = cls_ref[0, :, pl.ds(lo, wch)]
    codei = code_ref[0, :, pl.ds(lo, wch)]
    c = jnp.clip(x, 1e-4, 1.0 - 1e-4)
    t1 = (codei >= 0) & (cls_id_full[:, :wch] == codei)
    p = jnp.where(t1, c, 1.0 - c)
    af = jnp.where(t1, 0.25, jnp.where(codei == -1, 0.0, 0.75))
    val = af * jnp.square(1.0 - p) * (-jnp.log(p))
    if wch == _FCH:
      v = val
      w = _FCH // 2
      while w >= 128:
        v = v[:, :w] + v[:, w:2 * w]
        w //= 2
      acc = acc + v
    else:
      nfull = wch // 128
      for k in range(nfull):
        acc = acc + val[:, k * 128:(k + 1) * 128]
      rem = wch - nfull * 128
      if rem:
        acc = acc + jnp.concatenate(
            [val[:, nfull * 128:],
             jnp.zeros((_CBLK, 128 - rem), jnp.float32)], axis=1)

  @pl.when(ci == 0)
  def _init():
    out_ref[...] = jnp.zeros_like(out_ref)

  out_ref[0] = out_ref[0] + acc


def _focal_sums(cls_t, code3):
  return pl.pallas_call(
      _focal_body,
      grid=(_B, _NCB),
      in_specs=[
          pl.BlockSpec((1, _CBLK, _A), lambda b, ci: (b, ci, 0)),
          pl.BlockSpec((1, 1, _A), lambda b, ci: (b, 0, 0)),
      ],
      out_specs=pl.BlockSpec((1, _CBLK, 128), lambda b, ci: (b, 0, 0)),
      out_shape=jax.ShapeDtypeStruct((_B, _CBLK, 128), jnp.float32),
      compiler_params=pltpu.CompilerParams(
          dimension_semantics=("arbitrary", "arbitrary")),
  )(cls_t, code3)


@jax.jit
def kernel(regression, classification, anchors, gt_BB):
  f32 = jnp.float32
  anch_t = jnp.transpose(anchors[0].astype(f32), (1, 0))
  reg32 = jnp.transpose(regression.astype(f32), (2, 0, 1)).reshape(32, _A)
  cls_t = jnp.transpose(classification.astype(f32), (0, 2, 1))
  g = gt_BB.astype(f32).reshape(_NGT, 5)
  garea = (g[:, 2] - g[:, 0]) * (g[:, 3] - g[:, 1])
  gtf = jnp.concatenate(
      [g[:, 0:4].T, garea[None, :], g[:, 4][None, :],
       jnp.zeros((2, _NGT), f32)], axis=0)
  gtsf = jnp.broadcast_to(gtf.T.reshape(_NGT * 8)[:, None], (_NGT * 8, 16))
  gti = g[:, 4].astype(jnp.int32)

  code, bidx, nposp = _sc_assign(anch_t, gtsf, gti)
  part = _sc_regloss(anch_t, reg32, bidx, gtf)

  cls_acc = _focal_sums(cls_t, code.reshape(_B, 1, _A))
  cls_sums = cls_acc.sum(axis=(1, 2))

  npos = nposp.sum(axis=(0, 2))
  regs = part.sum(axis=(0, 2))
  np1 = jnp.maximum(npos, 1.0)
  cls_out = jnp.mean(cls_sums / np1, keepdims=True)
  reg_out = jnp.mean(jnp.where(npos > 0, regs / (np1 * 4.0), 0.0),
                     keepdims=True) * 50.0
  return cls_out, reg_out


# speedup vs baseline: 1.4638x; 1.4638x over previous
import functools

import jax
import jax.numpy as jnp
from jax import lax
from jax.experimental import pallas as pl
from jax.experimental.pallas import tpu as pltpu
from jax.experimental.pallas import tpu_sc as plsc

_B, _A, _C, _G = 8, 49104, 80, 20
_NTILES = 32
_CHUNK = 1536
_LCHUNK = _A - 31 * _CHUNK
_NVEC = _CHUNK // 16
_LNVEC = _LCHUNK // 16
_NGT = _B * _G
_CBLK = 8
_NCB = _C // _CBLK
_LN2 = 0.6931471805599453
_SQRT2 = 1.4142135623730951

_LOG_COEFS = (
    0.11589569107111188, -0.1885243878963612, 0.20655334597565636,
    -0.24886378324342587, 0.3329959787175392, -0.5000199301348872,
    1.0000040901688678, 6.432101458397899e-08,
)

_SC_PARAMS = pltpu.CompilerParams(use_tc_tiling_on_sc=False,
                                  needs_layout_passes=False)


def _mesh():
  return plsc.VectorSubcoreMesh(core_axis_name="c", subcore_axis_name="s",
                                num_cores=2, num_subcores=16)


def _sc_log(x):
  bits = plsc.bitcast(x, jnp.int32)
  e = ((bits >> 23) & 0xFF) - 127
  m = plsc.bitcast((bits & 0x7FFFFF) | 0x3F800000, jnp.float32)
  big = m > _SQRT2
  m = jnp.where(big, m * 0.5, m)
  ef = (e + jnp.where(big, 1, 0)).astype(jnp.float32)
  z = m - 1.0
  p = jnp.full_like(x, _LOG_COEFS[0])
  for c in _LOG_COEFS[1:]:
    p = p * z + c
  return ef * _LN2 + p


def _smooth_l1(d):
  d = jnp.abs(d)
  return jnp.where(d <= 1.0 / 9.0, 0.5 * 9.0 * d * d, d - 0.5 / 9.0)


def _tile_id():
  return lax.axis_index("s") * 2 + lax.axis_index("c")


def _assign_body(anch_hbm, gts_hbm, gti_hbm,
                 code_hbm, bidx_hbm, npos_hbm,
                 anch_v, gts_v, gti_v, code_v, bidx_v, acc_v):
  wid = _tile_id()
  last = wid == _NTILES - 1
  base = wid * _CHUNK

  @pl.when(jnp.logical_not(last))
  def _stage_full():
    pltpu.sync_copy(anch_hbm.at[:, pl.ds(base, _CHUNK)],
                    anch_v.at[:, pl.ds(0, _CHUNK)])

  @pl.when(last)
  def _stage_last():
    pltpu.sync_copy(anch_hbm.at[:, pl.ds(31 * _CHUNK, _LCHUNK)],
                    anch_v.at[:, pl.ds(0, _LCHUNK)])

  pltpu.sync_copy(gts_hbm, gts_v)
  pltpu.sync_copy(gti_hbm, gti_v)

  for b in range(_B):
    acc_v[b] = jnp.zeros((16,), jnp.float32)

  def process(o):
    a_x0 = anch_v[0, pl.ds(o, 16)]
    a_y0 = anch_v[1, pl.ds(o, 16)]
    a_x1 = anch_v[2, pl.ds(o, 16)]
    a_y1 = anch_v[3, pl.ds(o, 16)]
    a_area = (a_x1 - a_x0) * (a_y1 - a_y0)

    for b in range(_B):
      binter = jnp.full((16,), -1.0, jnp.float32)
      bdenom = jnp.full((16,), 1.0, jnp.float32)
      bidx = jnp.full((16,), b * _G, jnp.int32)
      for g in range(_G):
        k = b * _G + g
        gx0 = gts_v[k * 8 + 0]
        gy0 = gts_v[k * 8 + 1]
        gx1 = gts_v[k * 8 + 2]
        gy1 = gts_v[k * 8 + 3]
        gar = gts_v[k * 8 + 4]
        iw = jnp.maximum(
            jnp.minimum(a_x1, gx1) - jnp.maximum(a_x0, gx0), 0.0)
        ih = jnp.maximum(
            jnp.minimum(a_y1, gy1) - jnp.maximum(a_y0, gy0), 0.0)
        inter = iw * ih
        denom = (a_area + gar) - inter
        m = inter * bdenom > binter * denom
        binter = jnp.where(m, inter, binter)
        bdenom = jnp.where(m, denom, bdenom)
        bidx = jnp.where(m, jnp.full((16,), k, jnp.int32), bidx)

      pos = binter >= 0.5 * bdenom
      neg = binter < 0.4 * bdenom
      lab = plsc.load_gather(gti_v, [bidx])
      code = jnp.where(pos, lab, jnp.where(neg, -2, -1))
      code_v[b, pl.ds(o, 16)] = code
      bidx_v[b, pl.ds(o, 16)] = jnp.where(pos, bidx, -1)
      acc_v[b] = acc_v[b] + jnp.where(pos, 1.0, 0.0)

  nvec = jnp.where(last, _LNVEC, _NVEC)

  def body(i, carry):
    process(i * 16)
    return carry

  lax.fori_loop(0, nvec, body, 0)

  @pl.when(jnp.logical_not(last))
  def _out_full():
    for b in range(_B):
      pltpu.sync_copy(code_v.at[b, pl.ds(0, _CHUNK)],
                      code_hbm.at[b, pl.ds(base, _CHUNK)])
      pltpu.sync_copy(bidx_v.at[b, pl.ds(0, _CHUNK)],
                      bidx_hbm.at[b, pl.ds(base, _CHUNK)])

  @pl.when(last)
  def _out_last():
    for b in range(_B):
      pltpu.sync_copy(code_v.at[b, pl.ds(0, _LCHUNK)],
                      code_hbm.at[b, pl.ds(31 * _CHUNK, _LCHUNK)])
      pltpu.sync_copy(bidx_v.at[b, pl.ds(0, _LCHUNK)],
                      bidx_hbm.at[b, pl.ds(31 * _CHUNK, _LCHUNK)])

  pltpu.sync_copy(acc_v, npos_hbm.at[wid])


def _sc_assign(anch_t, gtsf, gti):
  fn = pl.kernel(
      _assign_body,
      out_type=(
          jax.ShapeDtypeStruct((_B, _A), jnp.int32),
          jax.ShapeDtypeStruct((_B, _A), jnp.int32),
          jax.ShapeDtypeStruct((_NTILES, _B, 16), jnp.float32),
      ),
      mesh=_mesh(),
      scratch_types=[
          pltpu.VMEM((4, _CHUNK), jnp.float32),
          pltpu.VMEM((_NGT * 8, 16), jnp.float32),
          pltpu.VMEM((_NGT,), jnp.int32),
          pltpu.VMEM((_B, _CHUNK), jnp.int32),
          pltpu.VMEM((_B, _CHUNK), jnp.int32),
          pltpu.VMEM((_B, 16), jnp.float32),
      ],
      compiler_params=_SC_PARAMS,
      name="assign",
  )
  return fn(anch_t, gtsf, gti)


def _regloss_body(anch_hbm, reg_hbm, bidx_hbm, gtf_hbm,
                  part_hbm,
                  anch_v, reg_v, bidx_v, gtf_v, acc_v):
  wid = _tile_id()
  last = wid == _NTILES - 1
  base = wid * _CHUNK

  @pl.when(jnp.logical_not(last))
  def _stage_full():
    pltpu.sync_copy(anch_hbm.at[:, pl.ds(base, _CHUNK)],
                    anch_v.at[:, pl.ds(0, _CHUNK)])
    pltpu.sync_copy(reg_hbm.at[:, pl.ds(base, _CHUNK)],
                    reg_v.at[:, pl.ds(0, _CHUNK)])
    for b in range(_B):
      pltpu.sync_copy(bidx_hbm.at[b, pl.ds(base, _CHUNK)],
                      bidx_v.at[b, pl.ds(0, _CHUNK)])

  @pl.when(last)
  def _stage_last():
    pltpu.sync_copy(anch_hbm.at[:, pl.ds(31 * _CHUNK, _LCHUNK)],
                    anch_v.at[:, pl.ds(0, _LCHUNK)])
    pltpu.sync_copy(reg_hbm.at[:, pl.ds(31 * _CHUNK, _LCHUNK)],
                    reg_v.at[:, pl.ds(0, _LCHUNK)])
    for b in range(_B):
      pltpu.sync_copy(bidx_hbm.at[b, pl.ds(31 * _CHUNK, _LCHUNK)],
                      bidx_v.at[b, pl.ds(0, _LCHUNK)])

  pltpu.sync_copy(gtf_hbm, gtf_v)

  for b in range(_B):
    acc_v[b] = jnp.zeros((16,), jnp.float32)

  col = [jnp.full((16,), c, jnp.int32) for c in range(4)]
  nvec = jnp.where(last, _LNVEC, _NVEC)

  def body(i, carry):
    o = i * 16
    a_x0 = anch_v[0, pl.ds(o, 16)]
    a_y0 = anch_v[1, pl.ds(o, 16)]
    a_x1 = anch_v[2, pl.ds(o, 16)]
    a_y1 = anch_v[3, pl.ds(o, 16)]
    aw = jnp.abs(a_x0 - a_x1)
    ah = jnp.abs(a_y0 - a_y1)
    actr_x = a_x0 + 0.5 * aw
    actr_y = a_y0 + 0.5 * ah
    inv_aw = 1.0 / aw
    inv_ah = 1.0 / ah

    for b in range(_B):
      bidx = bidx_v[b, pl.ds(o, 16)]
      pos = bidx >= 0
      idx = jnp.maximum(bidx, 0)
      gx0 = plsc.load_gather(gtf_v, [col[0], idx])
      gy0 = plsc.load_gather(gtf_v, [col[1], idx])
      gx1 = plsc.load_gather(gtf_v, [col[2], idx])
      gy1 = plsc.load_gather(gtf_v, [col[3], idx])
      gw0 = gx1 - gx0
      gh0 = gy1 - gy0
      gcx = gx0 + 0.5 * gw0
      gcy = gy0 + 0.5 * gh0
      gw = jnp.maximum(gw0, 1.0)
      gh = jnp.maximum(gh0, 1.0)
      tdx = (gcx - actr_x) * inv_aw
      tdy = (gcy - actr_y) * inv_ah
      tdw = _sc_log(gw * inv_aw)
      tdh = _sc_log(gh * inv_ah)
      r0 = reg_v[b, pl.ds(o, 16)]
      r1 = reg_v[8 + b, pl.ds(o, 16)]
      r2 = reg_v[16 + b, pl.ds(o, 16)]
      r3 = reg_v[24 + b, pl.ds(o, 16)]
      rl = (_smooth_l1(tdx - r0) + _smooth_l1(tdy - r1)
            + _smooth_l1(tdh - r2) + _smooth_l1(tdw - r3))
      acc_v[b] = acc_v[b] + jnp.where(pos, rl, 0.0)
    return carry

  lax.fori_loop(0, nvec, body, 0)
  pltpu.sync_copy(acc_v, part_hbm.at[wid])


def _sc_regloss(anch_t, reg32, bidx, gtf):
  fn = pl.kernel(
      _regloss_body,
      out_type=jax.ShapeDtypeStruct((_NTILES, _B, 16), jnp.float32),
      mesh=_mesh(),
      scratch_types=[
          pltpu.VMEM((4, _CHUNK), jnp.float32),
          pltpu.VMEM((32, _CHUNK), jnp.float32),
          pltpu.VMEM((_B, _CHUNK), jnp.int32),
          pltpu.VMEM((8, _NGT), jnp.float32),
          pltpu.VMEM((_B, 16), jnp.float32),
      ],
      compiler_params=_SC_PARAMS,
      name="regloss",
  )
  return fn(anch_t, reg32, bidx, gtf)


_FCH = 1024
_NFCH = (_A + _FCH - 1) // _FCH


def _focal_body(cls_ref, code_ref, out_ref):
  ci = pl.program_id(1)
  cls_id_full = (ci * _CBLK
                 + lax.broadcasted_iota(jnp.int32, (_CBLK, _FCH), 0))
  acc = jnp.zeros((_CBLK, 128), jnp.float32)
  for j in range(_NFCH):
    lo = j * _FCH
    wch = min(_FCH, _A - lo)
    x = cls_ref[0, :, pl.ds(lo, wch)]
    codei = code_ref[0, :, pl.ds(lo, wch)]
    c = jnp.clip(x, 1e-4, 1.0 - 1e-4)
    t1 = (codei >= 0) & (cls_id_full[:, :wch] == codei)
    p = jnp.where(t1, c, 1.0 - c)
    af = jnp.where(t1, 0.25, jnp.where(codei == -1, 0.0, 0.75))
    val = af * jnp.square(1.0 - p) * (-jnp.log(p))
    if wch == _FCH:
      v = val
      w = _FCH // 2
      while w >= 128:
        v = v[:, :w] + v[:, w:2 * w]
        w //= 2
      acc = acc + v
    else:
      nfull = wch // 128
      for k in range(nfull):
        acc = acc + val[:, k * 128:(k + 1) * 128]
      rem = wch - nfull * 128
      if rem:
        acc = acc + jnp.concatenate(
            [val[:, nfull * 128:],
             jnp.zeros((_CBLK, 128 - rem), jnp.float32)], axis=1)

  @pl.when(ci == 0)
  def _init():
    out_ref[...] = jnp.zeros_like(out_ref)

  out_ref[0] = out_ref[0] + acc


def _focal_sums(cls_t, code3):
  return pl.pallas_call(
      _focal_body,
      grid=(_B, _NCB),
      in_specs=[
          pl.BlockSpec((1, _CBLK, _A), lambda b, ci: (b, ci, 0)),
          pl.BlockSpec((1, 1, _A), lambda b, ci: (b, 0, 0)),
      ],
      out_specs=pl.BlockSpec((1, _CBLK, 128), lambda b, ci: (b, 0, 0)),
      out_shape=jax.ShapeDtypeStruct((_B, _CBLK, 128), jnp.float32),
      compiler_params=pltpu.CompilerParams(
          dimension_semantics=("arbitrary", "arbitrary")),
  )(cls_t, code3)


@jax.jit
def kernel(regression, classification, anchors, gt_BB):
  f32 = jnp.float32
  anch_t = jnp.transpose(anchors[0].astype(f32), (1, 0))
  reg32 = jnp.transpose(regression.astype(f32), (2, 0, 1)).reshape(32, _A)
  cls_t = jnp.transpose(classification.astype(f32), (0, 2, 1))
  g = gt_BB.astype(f32).reshape(_NGT, 5)
  garea = (g[:, 2] - g[:, 0]) * (g[:, 3] - g[:, 1])
  gtf = jnp.concatenate(
      [g[:, 0:4].T, garea[None, :], g[:, 4][None, :],
       jnp.zeros((2, _NGT), f32)], axis=0)
  gtsf = jnp.broadcast_to(gtf.T.reshape(_NGT * 8)[:, None], (_NGT * 8, 16))
  gti = g[:, 4].astype(jnp.int32)

  code, bidx, nposp = _sc_assign(anch_t, gtsf, gti)
  part = _sc_regloss(anch_t, reg32, bidx, gtf)

  cls_acc = _focal_sums(cls_t, code.reshape(_B, 1, _A))
  cls_sums = cls_acc.sum(axis=(1, 2))

  npos = nposp.sum(axis=(0, 2))
  regs = part.sum(axis=(0, 2))
  np1 = jnp.maximum(npos, 1.0)
  cls_out = jnp.mean(cls_sums / np1, keepdims=True)
  reg_out = jnp.mean(jnp.where(npos > 0, regs / (np1 * 4.0), 0.0),
                     keepdims=True) * 50.0
  return cls_out, reg_out


# speedup vs baseline: 1.6212x; 1.1075x over previous
import functools

import jax
import jax.numpy as jnp
from jax import lax
from jax.experimental import pallas as pl
from jax.experimental.pallas import tpu as pltpu
from jax.experimental.pallas import tpu_sc as plsc

_B, _A, _C, _G = 8, 49104, 80, 20
_NTILES = 32
_CHUNK = 1536
_LCHUNK = _A - 31 * _CHUNK
_NVEC = _CHUNK // 16
_LNVEC = _LCHUNK // 16
_NGT = _B * _G
_CBLK = 16
_NCB = _C // _CBLK
_LN2 = 0.6931471805599453
_SQRT2 = 1.4142135623730951

_LOG_COEFS = (
    0.11589569107111188, -0.1885243878963612, 0.20655334597565636,
    -0.24886378324342587, 0.3329959787175392, -0.5000199301348872,
    1.0000040901688678, 6.432101458397899e-08,
)

_SC_PARAMS = pltpu.CompilerParams(use_tc_tiling_on_sc=False,
                                  needs_layout_passes=False)


def _mesh():
  return plsc.VectorSubcoreMesh(core_axis_name="c", subcore_axis_name="s",
                                num_cores=2, num_subcores=16)


def _sc_log(x):
  bits = plsc.bitcast(x, jnp.int32)
  e = ((bits >> 23) & 0xFF) - 127
  m = plsc.bitcast((bits & 0x7FFFFF) | 0x3F800000, jnp.float32)
  big = m > _SQRT2
  m = jnp.where(big, m * 0.5, m)
  ef = (e + jnp.where(big, 1, 0)).astype(jnp.float32)
  z = m - 1.0
  p = jnp.full_like(x, _LOG_COEFS[0])
  for c in _LOG_COEFS[1:]:
    p = p * z + c
  return ef * _LN2 + p


def _smooth_l1(d):
  d = jnp.abs(d)
  return jnp.where(d <= 1.0 / 9.0, 0.5 * 9.0 * d * d, d - 0.5 / 9.0)


def _tile_id():
  return lax.axis_index("s") * 2 + lax.axis_index("c")


def _assign_body(anch_hbm, gts_hbm, gti_hbm,
                 code_hbm, bidx_hbm, npos_hbm,
                 anch_v, gts_v, gti_v, code_v, bidx_v, acc_v):
  wid = _tile_id()
  last = wid == _NTILES - 1
  base = wid * _CHUNK

  @pl.when(jnp.logical_not(last))
  def _stage_full():
    pltpu.sync_copy(anch_hbm.at[:, pl.ds(base, _CHUNK)],
                    anch_v.at[:, pl.ds(0, _CHUNK)])

  @pl.when(last)
  def _stage_last():
    pltpu.sync_copy(anch_hbm.at[:, pl.ds(31 * _CHUNK, _LCHUNK)],
                    anch_v.at[:, pl.ds(0, _LCHUNK)])

  pltpu.sync_copy(gts_hbm, gts_v)
  pltpu.sync_copy(gti_hbm, gti_v)

  for b in range(_B):
    acc_v[b] = jnp.zeros((16,), jnp.float32)

  def process(o):
    a_x0 = anch_v[0, pl.ds(o, 16)]
    a_y0 = anch_v[1, pl.ds(o, 16)]
    a_x1 = anch_v[2, pl.ds(o, 16)]
    a_y1 = anch_v[3, pl.ds(o, 16)]
    a_area = (a_x1 - a_x0) * (a_y1 - a_y0)

    for b in range(_B):
      binter = jnp.full((16,), -1.0, jnp.float32)
      bdenom = jnp.full((16,), 1.0, jnp.float32)
      bidx = jnp.full((16,), b * _G, jnp.int32)
      for g in range(_G):
        k = b * _G + g
        gx0 = gts_v[k * 8 + 0]
        gy0 = gts_v[k * 8 + 1]
        gx1 = gts_v[k * 8 + 2]
        gy1 = gts_v[k * 8 + 3]
        gar = gts_v[k * 8 + 4]
        iw = jnp.maximum(
            jnp.minimum(a_x1, gx1) - jnp.maximum(a_x0, gx0), 0.0)
        ih = jnp.maximum(
            jnp.minimum(a_y1, gy1) - jnp.maximum(a_y0, gy0), 0.0)
        inter = iw * ih
        denom = (a_area + gar) - inter
        m = inter * bdenom > binter * denom
        binter = jnp.where(m, inter, binter)
        bdenom = jnp.where(m, denom, bdenom)
        bidx = jnp.where(m, jnp.full((16,), k, jnp.int32), bidx)

      pos = binter >= 0.5 * bdenom
      neg = binter < 0.4 * bdenom
      lab = plsc.load_gather(gti_v, [bidx])
      code = jnp.where(pos, lab, jnp.where(neg, -2, -1))
      code_v[b, pl.ds(o, 16)] = code
      bidx_v[b, pl.ds(o, 16)] = jnp.where(pos, bidx, -1)
      acc_v[b] = acc_v[b] + jnp.where(pos, 1.0, 0.0)

  nvec = jnp.where(last, _LNVEC, _NVEC)

  def body(i, carry):
    process(i * 16)
    return carry

  lax.fori_loop(0, nvec, body, 0)

  @pl.when(jnp.logical_not(last))
  def _out_full():
    for b in range(_B):
      pltpu.sync_copy(code_v.at[b, pl.ds(0, _CHUNK)],
                      code_hbm.at[b, pl.ds(base, _CHUNK)])
      pltpu.sync_copy(bidx_v.at[b, pl.ds(0, _CHUNK)],
                      bidx_hbm.at[b, pl.ds(base, _CHUNK)])

  @pl.when(last)
  def _out_last():
    for b in range(_B):
      pltpu.sync_copy(code_v.at[b, pl.ds(0, _LCHUNK)],
                      code_hbm.at[b, pl.ds(31 * _CHUNK, _LCHUNK)])
      pltpu.sync_copy(bidx_v.at[b, pl.ds(0, _LCHUNK)],
                      bidx_hbm.at[b, pl.ds(31 * _CHUNK, _LCHUNK)])

  pltpu.sync_copy(acc_v, npos_hbm.at[wid])


def _sc_assign(anch_t, gtsf, gti):
  fn = pl.kernel(
      _assign_body,
      out_type=(
          jax.ShapeDtypeStruct((_B, _A), jnp.int32),
          jax.ShapeDtypeStruct((_B, _A), jnp.int32),
          jax.ShapeDtypeStruct((_NTILES, _B, 16), jnp.float32),
      ),
      mesh=_mesh(),
      scratch_types=[
          pltpu.VMEM((4, _CHUNK), jnp.float32),
          pltpu.VMEM((_NGT * 8, 16), jnp.float32),
          pltpu.VMEM((_NGT,), jnp.int32),
          pltpu.VMEM((_B, _CHUNK), jnp.int32),
          pltpu.VMEM((_B, _CHUNK), jnp.int32),
          pltpu.VMEM((_B, 16), jnp.float32),
      ],
      compiler_params=_SC_PARAMS,
      name="assign",
  )
  return fn(anch_t, gtsf, gti)


def _regloss_body(anch_hbm, reg_hbm, bidx_hbm, gtf_hbm,
                  part_hbm,
                  anch_v, reg_v, bidx_v, gtf_v, acc_v):
  wid = _tile_id()
  last = wid == _NTILES - 1
  base = wid * _CHUNK

  @pl.when(jnp.logical_not(last))
  def _stage_full():
    pltpu.sync_copy(anch_hbm.at[:, pl.ds(base, _CHUNK)],
                    anch_v.at[:, pl.ds(0, _CHUNK)])
    pltpu.sync_copy(reg_hbm.at[:, pl.ds(base, _CHUNK)],
                    reg_v.at[:, pl.ds(0, _CHUNK)])
    for b in range(_B):
      pltpu.sync_copy(bidx_hbm.at[b, pl.ds(base, _CHUNK)],
                      bidx_v.at[b, pl.ds(0, _CHUNK)])

  @pl.when(last)
  def _stage_last():
    pltpu.sync_copy(anch_hbm.at[:, pl.ds(31 * _CHUNK, _LCHUNK)],
                    anch_v.at[:, pl.ds(0, _LCHUNK)])
    pltpu.sync_copy(reg_hbm.at[:, pl.ds(31 * _CHUNK, _LCHUNK)],
                    reg_v.at[:, pl.ds(0, _LCHUNK)])
    for b in range(_B):
      pltpu.sync_copy(bidx_hbm.at[b, pl.ds(31 * _CHUNK, _LCHUNK)],
                      bidx_v.at[b, pl.ds(0, _LCHUNK)])

  pltpu.sync_copy(gtf_hbm, gtf_v)

  for b in range(_B):
    acc_v[b] = jnp.zeros((16,), jnp.float32)

  col = [jnp.full((16,), c, jnp.int32) for c in range(4)]
  nvec = jnp.where(last, _LNVEC, _NVEC)

  def body(i, carry):
    o = i * 16
    a_x0 = anch_v[0, pl.ds(o, 16)]
    a_y0 = anch_v[1, pl.ds(o, 16)]
    a_x1 = anch_v[2, pl.ds(o, 16)]
    a_y1 = anch_v[3, pl.ds(o, 16)]
    aw = jnp.abs(a_x0 - a_x1)
    ah = jnp.abs(a_y0 - a_y1)
    actr_x = a_x0 + 0.5 * aw
    actr_y = a_y0 + 0.5 * ah
    inv_aw = 1.0 / aw
    inv_ah = 1.0 / ah

    for b in range(_B):
      bidx = bidx_v[b, pl.ds(o, 16)]
      pos = bidx >= 0
      idx = jnp.maximum(bidx, 0)
      gx0 = plsc.load_gather(gtf_v, [col[0], idx])
      gy0 = plsc.load_gather(gtf_v, [col[1], idx])
      gx1 = plsc.load_gather(gtf_v, [col[2], idx])
      gy1 = plsc.load_gather(gtf_v, [col[3], idx])
      gw0 = gx1 - gx0
      gh0 = gy1 - gy0
      gcx = gx0 + 0.5 * gw0
      gcy = gy0 + 0.5 * gh0
      gw = jnp.maximum(gw0, 1.0)
      gh = jnp.maximum(gh0, 1.0)
      tdx = (gcx - actr_x) * inv_aw
      tdy = (gcy - actr_y) * inv_ah
      tdw = _sc_log(gw * inv_aw)
      tdh = _sc_log(gh * inv_ah)
      r0 = reg_v[b, pl.ds(o, 16)]
      r1 = reg_v[8 + b, pl.ds(o, 16)]
      r2 = reg_v[16 + b, pl.ds(o, 16)]
      r3 = reg_v[24 + b, pl.ds(o, 16)]
      rl = (_smooth_l1(tdx - r0) + _smooth_l1(tdy - r1)
            + _smooth_l1(tdh - r2) + _smooth_l1(tdw - r3))
      acc_v[b] = acc_v[b] + jnp.where(pos, rl, 0.0)
    return carry

  lax.fori_loop(0, nvec, body, 0)
  pltpu.sync_copy(acc_v, part_hbm.at[wid])


def _sc_regloss(anch_t, reg32, bidx, gtf):
  fn = pl.kernel(
      _regloss_body,
      out_type=jax.ShapeDtypeStruct((_NTILES, _B, 16), jnp.float32),
      mesh=_mesh(),
      scratch_types=[
          pltpu.VMEM((4, _CHUNK), jnp.float32),
          pltpu.VMEM((32, _CHUNK), jnp.float32),
          pltpu.VMEM((_B, _CHUNK), jnp.int32),
          pltpu.VMEM((8, _NGT), jnp.float32),
          pltpu.VMEM((_B, 16), jnp.float32),
      ],
      compiler_params=_SC_PARAMS,
      name="regloss",
  )
  return fn(anch_t, reg32, bidx, gtf)


_FCH = 1024
_NFCH = (_A + _FCH - 1) // _FCH


def _focal_body(cls_ref, code_ref, out_ref):
  ci = pl.program_id(1)
  cls_id_full = (ci * _CBLK
                 + lax.broadcasted_iota(jnp.int32, (_CBLK, _FCH), 0))
  acc = jnp.zeros((_CBLK, 128), jnp.float32)
  for j in range(_NFCH):
    lo = j * _FCH
    wch = min(_FCH, _A - lo)
    x = cls_ref[0, :, pl.ds(lo, wch)]
    codei = code_ref[0, :, pl.ds(lo, wch)]
    c = jnp.clip(x, 1e-4, 1.0 - 1e-4)
    t1 = (codei >= 0) & (cls_id_full[:, :wch] == codei)
    p = jnp.where(t1, c, 1.0 - c)
    af = jnp.where(t1, 0.25, jnp.where(codei == -1, 0.0, 0.75))
    val = af * jnp.square(1.0 - p) * (-jnp.log(p))
    if wch == _FCH:
      v = val
      w = _FCH // 2
      while w >= 128:
        v = v[:, :w] + v[:, w:2 * w]
        w //= 2
      acc = acc + v
    else:
      nfull = wch // 128
      for k in range(nfull):
        acc = acc + val[:, k * 128:(k + 1) * 128]
      rem = wch - nfull * 128
      if rem:
        acc = acc + jnp.concatenate(
            [val[:, nfull * 128:],
             jnp.zeros((_CBLK, 128 - rem), jnp.float32)], axis=1)

  @pl.when(ci == 0)
  def _init():
    out_ref[...] = jnp.zeros_like(out_ref)

  out_ref[0] = out_ref[0] + acc


def _focal_sums(cls_t, code3):
  return pl.pallas_call(
      _focal_body,
      grid=(_B, _NCB),
      in_specs=[
          pl.BlockSpec((1, _CBLK, _A), lambda b, ci: (b, ci, 0)),
          pl.BlockSpec((1, 1, _A), lambda b, ci: (b, 0, 0)),
      ],
      out_specs=pl.BlockSpec((1, _CBLK, 128), lambda b, ci: (b, 0, 0)),
      out_shape=jax.ShapeDtypeStruct((_B, _CBLK, 128), jnp.float32),
      compiler_params=pltpu.CompilerParams(
          dimension_semantics=("arbitrary", "arbitrary")),
  )(cls_t, code3)


@jax.jit
def kernel(regression, classification, anchors, gt_BB):
  f32 = jnp.float32
  anch_t = jnp.transpose(anchors[0].astype(f32), (1, 0))
  reg32 = jnp.transpose(regression.astype(f32), (2, 0, 1)).reshape(32, _A)
  cls_t = jnp.transpose(classification.astype(f32), (0, 2, 1))
  g = gt_BB.astype(f32).reshape(_NGT, 5)
  garea = (g[:, 2] - g[:, 0]) * (g[:, 3] - g[:, 1])
  gtf = jnp.concatenate(
      [g[:, 0:4].T, garea[None, :], g[:, 4][None, :],
       jnp.zeros((2, _NGT), f32)], axis=0)
  gtsf = jnp.broadcast_to(gtf.T.reshape(_NGT * 8)[:, None], (_NGT * 8, 16))
  gti = g[:, 4].astype(jnp.int32)

  code, bidx, nposp = _sc_assign(anch_t, gtsf, gti)
  part = _sc_regloss(anch_t, reg32, bidx, gtf)

  cls_acc = _focal_sums(cls_t, code.reshape(_B, 1, _A))
  cls_sums = cls_acc.sum(axis=(1, 2))

  npos = nposp.sum(axis=(0, 2))
  regs = part.sum(axis=(0, 2))
  np1 = jnp.maximum(npos, 1.0)
  cls_out = jnp.mean(cls_sums / np1, keepdims=True)
  reg_out = jnp.mean(jnp.where(npos > 0, regs / (np1 * 4.0), 0.0),
                     keepdims=True) * 50.0
  return cls_out, reg_out


# speedup vs baseline: 1.6434x; 1.0137x over previous
import functools

import jax
import jax.numpy as jnp
from jax import lax
from jax.experimental import pallas as pl
from jax.experimental.pallas import tpu as pltpu
from jax.experimental.pallas import tpu_sc as plsc

_B, _A, _C, _G = 8, 49104, 80, 20
_NTILES = 32
_CHUNK = 1536
_LCHUNK = _A - 31 * _CHUNK
_NVEC = _CHUNK // 16
_LNVEC = _LCHUNK // 16
_NGT = _B * _G
_CBLK = 16
_NCB = _C // _CBLK
_LN2 = 0.6931471805599453
_SQRT2 = 1.4142135623730951

_LOG_COEFS = (
    0.11589569107111188, -0.1885243878963612, 0.20655334597565636,
    -0.24886378324342587, 0.3329959787175392, -0.5000199301348872,
    1.0000040901688678, 6.432101458397899e-08,
)

_SC_PARAMS = pltpu.CompilerParams(use_tc_tiling_on_sc=False,
                                  needs_layout_passes=False)


def _mesh():
  return plsc.VectorSubcoreMesh(core_axis_name="c", subcore_axis_name="s",
                                num_cores=2, num_subcores=16)


def _sc_log(x):
  bits = plsc.bitcast(x, jnp.int32)
  e = ((bits >> 23) & 0xFF) - 127
  m = plsc.bitcast((bits & 0x7FFFFF) | 0x3F800000, jnp.float32)
  big = m > _SQRT2
  m = jnp.where(big, m * 0.5, m)
  ef = (e + jnp.where(big, 1, 0)).astype(jnp.float32)
  z = m - 1.0
  p = jnp.full_like(x, _LOG_COEFS[0])
  for c in _LOG_COEFS[1:]:
    p = p * z + c
  return ef * _LN2 + p


def _smooth_l1(d):
  d = jnp.abs(d)
  return jnp.where(d <= 1.0 / 9.0, 0.5 * 9.0 * d * d, d - 0.5 / 9.0)


def _tile_id():
  return lax.axis_index("s") * 2 + lax.axis_index("c")


def _assign_body(anch_hbm, gts_hbm, gti_hbm,
                 code_hbm, bidx_hbm, npos_hbm,
                 anch_v, gts_v, gti_v, code_v, bidx_v, acc_v):
  wid = _tile_id()
  last = wid == _NTILES - 1
  base = wid * _CHUNK

  @pl.when(jnp.logical_not(last))
  def _stage_full():
    pltpu.sync_copy(anch_hbm.at[:, pl.ds(base, _CHUNK)],
                    anch_v.at[:, pl.ds(0, _CHUNK)])

  @pl.when(last)
  def _stage_last():
    pltpu.sync_copy(anch_hbm.at[:, pl.ds(31 * _CHUNK, _LCHUNK)],
                    anch_v.at[:, pl.ds(0, _LCHUNK)])

  pltpu.sync_copy(gts_hbm, gts_v)
  pltpu.sync_copy(gti_hbm, gti_v)

  for b in range(_B):
    acc_v[b] = jnp.zeros((16,), jnp.float32)

  def process(o):
    a_x0 = anch_v[0, pl.ds(o, 16)]
    a_y0 = anch_v[1, pl.ds(o, 16)]
    a_x1 = anch_v[2, pl.ds(o, 16)]
    a_y1 = anch_v[3, pl.ds(o, 16)]
    a_area = (a_x1 - a_x0) * (a_y1 - a_y0)

    for b in range(_B):
      binter = jnp.full((16,), -1.0, jnp.float32)
      bdenom = jnp.full((16,), 1.0, jnp.float32)
      bidx = jnp.full((16,), b * _G, jnp.int32)
      for g in range(_G):
        k = b * _G + g
        gx0 = gts_v[k * 8 + 0]
        gy0 = gts_v[k * 8 + 1]
        gx1 = gts_v[k * 8 + 2]
        gy1 = gts_v[k * 8 + 3]
        gar = gts_v[k * 8 + 4]
        iw = jnp.maximum(
            jnp.minimum(a_x1, gx1) - jnp.maximum(a_x0, gx0), 0.0)
        ih = jnp.maximum(
            jnp.minimum(a_y1, gy1) - jnp.maximum(a_y0, gy0), 0.0)
        inter = iw * ih
        denom = (a_area + gar) - inter
        m = inter * bdenom > binter * denom
        binter = jnp.where(m, inter, binter)
        bdenom = jnp.where(m, denom, bdenom)
        bidx = jnp.where(m, jnp.full((16,), k, jnp.int32), bidx)

      pos = binter >= 0.5 * bdenom
      neg = binter < 0.4 * bdenom
      lab = plsc.load_gather(gti_v, [bidx])
      code = jnp.where(pos, lab, jnp.where(neg, -2, -1))
      code_v[b, pl.ds(o, 16)] = code
      bidx_v[b, pl.ds(o, 16)] = jnp.where(pos, bidx, -1)
      acc_v[b] = acc_v[b] + jnp.where(pos, 1.0, 0.0)

  nvec = jnp.where(last, _LNVEC, _NVEC)

  def body(i, carry):
    process(i * 16)
    return carry

  lax.fori_loop(0, nvec, body, 0)

  @pl.when(jnp.logical_not(last))
  def _out_full():
    for b in range(_B):
      pltpu.sync_copy(code_v.at[b, pl.ds(0, _CHUNK)],
                      code_hbm.at[b, pl.ds(base, _CHUNK)])
      pltpu.sync_copy(bidx_v.at[b, pl.ds(0, _CHUNK)],
                      bidx_hbm.at[b, pl.ds(base, _CHUNK)])

  @pl.when(last)
  def _out_last():
    for b in range(_B):
      pltpu.sync_copy(code_v.at[b, pl.ds(0, _LCHUNK)],
                      code_hbm.at[b, pl.ds(31 * _CHUNK, _LCHUNK)])
      pltpu.sync_copy(bidx_v.at[b, pl.ds(0, _LCHUNK)],
                      bidx_hbm.at[b, pl.ds(31 * _CHUNK, _LCHUNK)])

  pltpu.sync_copy(acc_v, npos_hbm.at[wid])


def _sc_assign(anch_t, gtsf, gti):
  fn = pl.kernel(
      _assign_body,
      out_type=(
          jax.ShapeDtypeStruct((_B, _A), jnp.int32),
          jax.ShapeDtypeStruct((_B, _A), jnp.int32),
          jax.ShapeDtypeStruct((_NTILES, _B, 16), jnp.float32),
      ),
      mesh=_mesh(),
      scratch_types=[
          pltpu.VMEM((4, _CHUNK), jnp.float32),
          pltpu.VMEM((_NGT * 8, 16), jnp.float32),
          pltpu.VMEM((_NGT,), jnp.int32),
          pltpu.VMEM((_B, _CHUNK), jnp.int32),
          pltpu.VMEM((_B, _CHUNK), jnp.int32),
          pltpu.VMEM((_B, 16), jnp.float32),
      ],
      compiler_params=_SC_PARAMS,
      name="assign",
  )
  return fn(anch_t, gtsf, gti)


def _regloss_body(anch_hbm, reg_hbm, bidx_hbm, gtf_hbm,
                  part_hbm,
                  anch_v, reg_v, bidx_v, gtf_v, acc_v):
  wid = _tile_id()
  last = wid == _NTILES - 1
  base = wid * _CHUNK

  @pl.when(jnp.logical_not(last))
  def _stage_full():
    pltpu.sync_copy(anch_hbm.at[:, pl.ds(base, _CHUNK)],
                    anch_v.at[:, pl.ds(0, _CHUNK)])
    pltpu.sync_copy(reg_hbm.at[:, pl.ds(base, _CHUNK)],
                    reg_v.at[:, pl.ds(0, _CHUNK)])
    for b in range(_B):
      pltpu.sync_copy(bidx_hbm.at[b, pl.ds(base, _CHUNK)],
                      bidx_v.at[b, pl.ds(0, _CHUNK)])

  @pl.when(last)
  def _stage_last():
    pltpu.sync_copy(anch_hbm.at[:, pl.ds(31 * _CHUNK, _LCHUNK)],
                    anch_v.at[:, pl.ds(0, _LCHUNK)])
    pltpu.sync_copy(reg_hbm.at[:, pl.ds(31 * _CHUNK, _LCHUNK)],
                    reg_v.at[:, pl.ds(0, _LCHUNK)])
    for b in range(_B):
      pltpu.sync_copy(bidx_hbm.at[b, pl.ds(31 * _CHUNK, _LCHUNK)],
                      bidx_v.at[b, pl.ds(0, _LCHUNK)])

  pltpu.sync_copy(gtf_hbm, gtf_v)

  for b in range(_B):
    acc_v[b] = jnp.zeros((16,), jnp.float32)

  col = [jnp.full((16,), c, jnp.int32) for c in range(4)]
  nvec = jnp.where(last, _LNVEC, _NVEC)

  def body(i, carry):
    o = i * 16
    a_x0 = anch_v[0, pl.ds(o, 16)]
    a_y0 = anch_v[1, pl.ds(o, 16)]
    a_x1 = anch_v[2, pl.ds(o, 16)]
    a_y1 = anch_v[3, pl.ds(o, 16)]
    aw = jnp.abs(a_x0 - a_x1)
    ah = jnp.abs(a_y0 - a_y1)
    actr_x = a_x0 + 0.5 * aw
    actr_y = a_y0 + 0.5 * ah
    inv_aw = 1.0 / aw
    inv_ah = 1.0 / ah

    for b in range(_B):
      bidx = bidx_v[b, pl.ds(o, 16)]
      pos = bidx >= 0
      idx = jnp.maximum(bidx, 0)
      gx0 = plsc.load_gather(gtf_v, [col[0], idx])
      gy0 = plsc.load_gather(gtf_v, [col[1], idx])
      gx1 = plsc.load_gather(gtf_v, [col[2], idx])
      gy1 = plsc.load_gather(gtf_v, [col[3], idx])
      gw0 = gx1 - gx0
      gh0 = gy1 - gy0
      gcx = gx0 + 0.5 * gw0
      gcy = gy0 + 0.5 * gh0
      gw = jnp.maximum(gw0, 1.0)
      gh = jnp.maximum(gh0, 1.0)
      tdx = (gcx - actr_x) * inv_aw
      tdy = (gcy - actr_y) * inv_ah
      tdw = _sc_log(gw * inv_aw)
      tdh = _sc_log(gh * inv_ah)
      r0 = reg_v[b, pl.ds(o, 16)]
      r1 = reg_v[8 + b, pl.ds(o, 16)]
      r2 = reg_v[16 + b, pl.ds(o, 16)]
      r3 = reg_v[24 + b, pl.ds(o, 16)]
      rl = (_smooth_l1(tdx - r0) + _smooth_l1(tdy - r1)
            + _smooth_l1(tdh - r2) + _smooth_l1(tdw - r3))
      acc_v[b] = acc_v[b] + jnp.where(pos, rl, 0.0)
    return carry

  lax.fori_loop(0, nvec, body, 0)
  pltpu.sync_copy(acc_v, part_hbm.at[wid])


def _sc_regloss(anch_t, reg32, bidx, gtf):
  fn = pl.kernel(
      _regloss_body,
      out_type=jax.ShapeDtypeStruct((_NTILES, _B, 16), jnp.float32),
      mesh=_mesh(),
      scratch_types=[
          pltpu.VMEM((4, _CHUNK), jnp.float32),
          pltpu.VMEM((32, _CHUNK), jnp.float32),
          pltpu.VMEM((_B, _CHUNK), jnp.int32),
          pltpu.VMEM((8, _NGT), jnp.float32),
          pltpu.VMEM((_B, 16), jnp.float32),
      ],
      compiler_params=_SC_PARAMS,
      name="regloss",
  )
  return fn(anch_t, reg32, bidx, gtf)


_FCH = 512
_NFCH = (_A + _FCH - 1) // _FCH


def _focal_body(cls_ref, code_ref, out_ref):
  ci = pl.program_id(1)
  cls_id_full = (ci * _CBLK
                 + lax.broadcasted_iota(jnp.int32, (_CBLK, _FCH), 0))
  acc = jnp.zeros((_CBLK, 128), jnp.float32)
  for j in range(_NFCH):
    lo = j * _FCH
    wch = min(_FCH, _A - lo)
    x = cls_ref[0, :, pl.ds(lo, wch)]
    codei = code_ref[0, :, pl.ds(lo, wch)]
    c = jnp.clip(x, 1e-4, 1.0 - 1e-4)
    t1 = (codei >= 0) & (cls_id_full[:, :wch] == codei)
    p = jnp.where(t1, c, 1.0 - c)
    af = jnp.where(t1, 0.25, jnp.where(codei == -1, 0.0, 0.75))
    val = af * jnp.square(1.0 - p) * (-jnp.log(p))
    if wch == _FCH:
      v = val
      w = _FCH // 2
      while w >= 128:
        v = v[:, :w] + v[:, w:2 * w]
        w //= 2
      acc = acc + v
    else:
      nfull = wch // 128
      for k in range(nfull):
        acc = acc + val[:, k * 128:(k + 1) * 128]
      rem = wch - nfull * 128
      if rem:
        acc = acc + jnp.concatenate(
            [val[:, nfull * 128:],
             jnp.zeros((_CBLK, 128 - rem), jnp.float32)], axis=1)

  @pl.when(ci == 0)
  def _init():
    out_ref[...] = jnp.zeros_like(out_ref)

  out_ref[0] = out_ref[0] + acc


def _focal_sums(cls_t, code3):
  return pl.pallas_call(
      _focal_body,
      grid=(_B, _NCB),
      in_specs=[
          pl.BlockSpec((1, _CBLK, _A), lambda b, ci: (b, ci, 0)),
          pl.BlockSpec((1, 1, _A), lambda b, ci: (b, 0, 0)),
      ],
      out_specs=pl.BlockSpec((1, _CBLK, 128), lambda b, ci: (b, 0, 0)),
      out_shape=jax.ShapeDtypeStruct((_B, _CBLK, 128), jnp.float32),
      compiler_params=pltpu.CompilerParams(
          dimension_semantics=("arbitrary", "arbitrary")),
  )(cls_t, code3)


@jax.jit
def kernel(regression, classification, anchors, gt_BB):
  f32 = jnp.float32
  anch_t = jnp.transpose(anchors[0].astype(f32), (1, 0))
  reg32 = jnp.transpose(regression.astype(f32), (2, 0, 1)).reshape(32, _A)
  cls_t = jnp.transpose(classification.astype(f32), (0, 2, 1))
  g = gt_BB.astype(f32).reshape(_NGT, 5)
  garea = (g[:, 2] - g[:, 0]) * (g[:, 3] - g[:, 1])
  gtf = jnp.concatenate(
      [g[:, 0:4].T, garea[None, :], g[:, 4][None, :],
       jnp.zeros((2, _NGT), f32)], axis=0)
  gtsf = jnp.broadcast_to(gtf.T.reshape(_NGT * 8)[:, None], (_NGT * 8, 16))
  gti = g[:, 4].astype(jnp.int32)

  code, bidx, nposp = _sc_assign(anch_t, gtsf, gti)
  part = _sc_regloss(anch_t, reg32, bidx, gtf)

  cls_acc = _focal_sums(cls_t, code.reshape(_B, 1, _A))
  cls_sums = cls_acc.sum(axis=(1, 2))

  npos = nposp.sum(axis=(0, 2))
  regs = part.sum(axis=(0, 2))
  np1 = jnp.maximum(npos, 1.0)
  cls_out = jnp.mean(cls_sums / np1, keepdims=True)
  reg_out = jnp.mean(jnp.where(npos > 0, regs / (np1 * 4.0), 0.0),
                     keepdims=True) * 50.0
  return cls_out, reg_out
